# baseline (speedup 1.0000x reference)
import functools

import jax
import jax.numpy as jnp
from jax import lax
from jax.experimental import pallas as pl
from jax.experimental.pallas import tpu as pltpu
from jax.experimental.pallas import tpu_sc as plsc

B, N, D, H = 4, 256, 128, 128
ALPHA = 0.2
K_NN = 8
NEG_BIG = -9e15

ROWS = B * N
NWORKERS = 32
RPW = ROWS // NWORKERS
NCHUNK = N // 16


def _dist_body(z_ref, d_ref):
    for b in range(B):
        z = z_ref[b]
        G = lax.dot_general(z, z, (((1,), (1,)), ((), ())),
                            preferred_element_type=jnp.float32,
                            precision=lax.Precision.HIGHEST)
        s = jnp.sum(z * z, axis=1)
        d = s[:, None] + s[None, :] - 2.0 * G
        iota_i = lax.broadcasted_iota(jnp.int32, (N, N), 0)
        iota_j = lax.broadcasted_iota(jnp.int32, (N, N), 1)
        d_ref[pl.ds(b * N, N), :] = jnp.where(iota_i == iota_j, 1e6, d)


def _topk_sc_body(d_hbm, adj_hbm, rows_v, adj_v):
    wid = lax.axis_index("s") * 2 + lax.axis_index("c")
    base = wid * RPW
    pltpu.sync_copy(d_hbm.at[pl.ds(base, RPW)], rows_v)
    lane = lax.broadcasted_iota(jnp.int32, (16,), 0)
    zeros16 = jnp.zeros((16,), jnp.float32)
    ones16 = jnp.ones((16,), jnp.float32)
    low8 = lane < K_NN

    @plsc.parallel_loop(0, RPW, unroll=4)
    def row_body(r):
        nodes = []
        for c in range(NCHUNK):
            k = rows_v[r, pl.ds(c * 16, 16)]
            nodes.append(plsc.sort_key_val(k, lane + c * 16))
        while len(nodes) > 1:
            nxt = []
            for (ka, ia), (kb, ib) in zip(nodes[::2], nodes[1::2]):
                kbr = lax.rev(kb, (0,))
                ibr = lax.rev(ib, (0,))
                lt = ka <= kbr
                ck = jnp.where(lt, ka, kbr)
                ci = jnp.where(lt, ia, ibr)
                nxt.append(plsc.sort_key_val(ck, ci))
            nodes = nxt
        for c in range(NCHUNK):
            adj_v[r, pl.ds(c * 16, 16)] = zeros16
        plsc.store_scatter(adj_v, [jnp.full((16,), r, jnp.int32), nodes[0][1]],
                           ones16, mask=low8)

    pltpu.sync_copy(adj_v, adj_hbm.at[pl.ds(base, RPW)])


@functools.partial(
    pl.kernel,
    mesh=plsc.VectorSubcoreMesh(core_axis_name="c", subcore_axis_name="s"),
    out_type=jax.ShapeDtypeStruct((ROWS, N), jnp.float32),
    compiler_params=pltpu.CompilerParams(needs_layout_passes=False),
    scratch_types=[
        pltpu.VMEM((RPW, N), jnp.float32),
        pltpu.VMEM((RPW, N), jnp.float32),
    ],
)
def _topk_sc(d_hbm, adj_hbm, rows_v, adj_v):
    _topk_sc_body(d_hbm, adj_hbm, rows_v, adj_v)


def _gat_layer(x, att_bias3, W, a1c, a2c, g, be, apply_relu, ones_h):
    Wh = lax.dot_general(x, W, (((1,), (1,)), ((), ())),
                         preferred_element_type=jnp.float32)
    f1 = lax.dot_general(Wh, a1c, (((1,), (0,)), ((), ())),
                         preferred_element_type=jnp.float32)
    f2 = jnp.stack([
        lax.dot_general(a2c, Wh[b * N:(b + 1) * N], (((1,), (1,)), ((), ())),
                        preferred_element_type=jnp.float32)
        for b in range(B)])
    e = f1.reshape(B, N, 1) + f2
    e = jnp.where(e >= 0, e, ALPHA * e)
    att = e + att_bias3
    m = jnp.max(att, axis=2, keepdims=True)
    ex = jnp.exp(att - m)
    p = ex / jnp.sum(ex, axis=2, keepdims=True)
    h = jnp.concatenate([
        lax.dot_general(p[b], Wh[b * N:(b + 1) * N], (((1,), (0,)), ((), ())),
                        preferred_element_type=jnp.float32)
        for b in range(B)])
    mu = lax.dot_general(h, ones_h, (((1,), (0,)), ((), ())),
                         preferred_element_type=jnp.float32) * (1.0 / H)
    hc = h - mu
    var = lax.dot_general(hc * hc, ones_h, (((1,), (0,)), ((), ())),
                          preferred_element_type=jnp.float32) * (1.0 / H)
    y = hc / jnp.sqrt(var + 1e-5) * g + be
    if apply_relu:
        y = jnp.maximum(y, 0.0)
    return x + y


def _gat_body(adj_ref, z_ref, Win_ref, bin_ref, W0_ref, a0c1_ref, a0c2_ref,
              g0_ref, be0_ref, W1_ref, a1c1_ref, a1c2_ref, g1_ref, be1_ref,
              out_ref):
    iota_i = lax.broadcasted_iota(jnp.int32, (N, N), 0)
    iota_j = lax.broadcasted_iota(jnp.int32, (N, N), 1)
    eye = jnp.where(iota_i == iota_j, 1.0, 0.0)
    ones_h = jnp.ones((H, 1), jnp.float32)
    S = jnp.stack([
        adj_ref[pl.ds(b * N, N), :]
        + lax.dot_general(adj_ref[pl.ds(b * N, N), :], eye,
                          (((0,), (0,)), ((), ())),
                          preferred_element_type=jnp.float32)
        for b in range(B)])
    att_bias3 = jnp.where(S > 0, 0.0, NEG_BIG)
    x = lax.dot_general(z_ref[...], Win_ref[...], (((1,), (1,)), ((), ())),
                        preferred_element_type=jnp.float32) + bin_ref[...]
    x = _gat_layer(x, att_bias3, W0_ref[...], a0c1_ref[...], a0c2_ref[...],
                   g0_ref[...], be0_ref[...], True, ones_h)
    x = _gat_layer(x, att_bias3, W1_ref[...], a1c1_ref[...], a1c2_ref[...],
                   g1_ref[...], be1_ref[...], False, ones_h)
    out_ref[...] = x.reshape(B, N, H)


@jax.jit
def _run(z, W_in, b_in, W0, a0, g0, be0, W1, a1, g1, be1):
    d = pl.pallas_call(
        _dist_body,
        out_shape=jax.ShapeDtypeStruct((ROWS, N), jnp.float32),
    )(z)

    adj = _topk_sc(d)

    return pl.pallas_call(
        _gat_body,
        out_shape=jax.ShapeDtypeStruct((B, N, H), jnp.float32),
    )(adj, z.reshape(ROWS, D), W_in, b_in.reshape(1, H),
      W0, a0[0, :H].reshape(H, 1), a0[0, H:].reshape(1, H),
      g0.reshape(1, H), be0.reshape(1, H),
      W1, a1[0, :H].reshape(H, 1), a1[0, H:].reshape(1, H),
      g1.reshape(1, H), be1.reshape(1, H))


def kernel(z, W_in, b_in, W0, a0, g0, be0, W1, a1, g1, be1):
    return _run(z, W_in, b_in, W0, a0, g0, be0, W1, a1, g1, be1)


# speedup vs baseline: 1.0452x; 1.0452x over previous
import functools

import jax
import jax.numpy as jnp
from jax import lax
from jax.experimental import pallas as pl
from jax.experimental.pallas import tpu as pltpu
from jax.experimental.pallas import tpu_sc as plsc

B, N, D, H = 4, 256, 128, 128
ALPHA = 0.2
K_NN = 8
NEG_BIG = -9e15

ROWS = B * N
NWORKERS = 32
RPW = ROWS // NWORKERS
NCHUNK = N // 16


def _dist_body(z_ref, d_ref):
    for b in range(B):
        z = z_ref[b]
        G = lax.dot_general(z, z, (((1,), (1,)), ((), ())),
                            preferred_element_type=jnp.float32,
                            precision=lax.Precision.HIGHEST)
        s = jnp.sum(z * z, axis=1)
        d = s[:, None] + s[None, :] - 2.0 * G
        iota_i = lax.broadcasted_iota(jnp.int32, (N, N), 0)
        iota_j = lax.broadcasted_iota(jnp.int32, (N, N), 1)
        d_ref[pl.ds(b * N, N), :] = jnp.where(iota_i == iota_j, 1e6, d)


def _topk_sc_body(d_hbm, adj_hbm, rows_v, adj_v):
    wid = lax.axis_index("s") * 2 + lax.axis_index("c")
    base = wid * RPW
    pltpu.sync_copy(d_hbm.at[pl.ds(base, RPW)], rows_v)
    lane = lax.broadcasted_iota(jnp.int32, (16,), 0)
    zeros16 = jnp.zeros((16,), jnp.float32)
    ones16 = jnp.ones((16,), jnp.float32)
    low8 = lane < K_NN

    @plsc.parallel_loop(0, RPW, unroll=2)
    def row_body(r):
        nodes = []
        for c in range(NCHUNK):
            k = rows_v[r, pl.ds(c * 16, 16)]
            nodes.append(plsc.sort_key_val(k, lane + c * 16))
        while len(nodes) > 1:
            nxt = []
            for (ka, ia), (kb, ib) in zip(nodes[::2], nodes[1::2]):
                kbr = lax.rev(kb, (0,))
                ibr = lax.rev(ib, (0,))
                lt = ka <= kbr
                ck = jnp.where(lt, ka, kbr)
                ci = jnp.where(lt, ia, ibr)
                nxt.append(plsc.sort_key_val(ck, ci))
            nodes = nxt
        for c in range(NCHUNK):
            adj_v[r, pl.ds(c * 16, 16)] = zeros16
        plsc.store_scatter(adj_v, [jnp.full((16,), r, jnp.int32), nodes[0][1]],
                           ones16, mask=low8)

    pltpu.sync_copy(adj_v, adj_hbm.at[pl.ds(base, RPW)])


@functools.partial(
    pl.kernel,
    mesh=plsc.VectorSubcoreMesh(core_axis_name="c", subcore_axis_name="s"),
    out_type=jax.ShapeDtypeStruct((ROWS, N), jnp.float32),
    compiler_params=pltpu.CompilerParams(needs_layout_passes=False),
    scratch_types=[
        pltpu.VMEM((RPW, N), jnp.float32),
        pltpu.VMEM((RPW, N), jnp.float32),
    ],
)
def _topk_sc(d_hbm, adj_hbm, rows_v, adj_v):
    _topk_sc_body(d_hbm, adj_hbm, rows_v, adj_v)


def _gat_layer(x, att_bias3, W, a1c, a2c, g, be, apply_relu, ones_h):
    Wh = lax.dot_general(x, W, (((1,), (1,)), ((), ())),
                         preferred_element_type=jnp.float32)
    f1 = lax.dot_general(Wh, a1c, (((1,), (0,)), ((), ())),
                         preferred_element_type=jnp.float32)
    f2 = jnp.stack([
        lax.dot_general(a2c, Wh[b * N:(b + 1) * N], (((1,), (1,)), ((), ())),
                        preferred_element_type=jnp.float32)
        for b in range(B)])
    e = f1.reshape(B, N, 1) + f2
    e = jnp.where(e >= 0, e, ALPHA * e)
    att = e + att_bias3
    m = jnp.max(att, axis=2, keepdims=True)
    ex = jnp.exp(att - m)
    p = ex / jnp.sum(ex, axis=2, keepdims=True)
    h = jnp.concatenate([
        lax.dot_general(p[b], Wh[b * N:(b + 1) * N], (((1,), (0,)), ((), ())),
                        preferred_element_type=jnp.float32)
        for b in range(B)])
    mu = lax.dot_general(h, ones_h, (((1,), (0,)), ((), ())),
                         preferred_element_type=jnp.float32) * (1.0 / H)
    hc = h - mu
    var = lax.dot_general(hc * hc, ones_h, (((1,), (0,)), ((), ())),
                          preferred_element_type=jnp.float32) * (1.0 / H)
    y = hc / jnp.sqrt(var + 1e-5) * g + be
    if apply_relu:
        y = jnp.maximum(y, 0.0)
    return x + y


def _gat_body(adj_ref, z_ref, Win_ref, bin_ref, W0_ref, a0c1_ref, a0c2_ref,
              g0_ref, be0_ref, W1_ref, a1c1_ref, a1c2_ref, g1_ref, be1_ref,
              out_ref):
    iota_i = lax.broadcasted_iota(jnp.int32, (N, N), 0)
    iota_j = lax.broadcasted_iota(jnp.int32, (N, N), 1)
    eye = jnp.where(iota_i == iota_j, 1.0, 0.0)
    ones_h = jnp.ones((H, 1), jnp.float32)
    S = jnp.stack([
        adj_ref[pl.ds(b * N, N), :]
        + lax.dot_general(adj_ref[pl.ds(b * N, N), :], eye,
                          (((0,), (0,)), ((), ())),
                          preferred_element_type=jnp.float32)
        for b in range(B)])
    att_bias3 = jnp.where(S > 0, 0.0, NEG_BIG)
    x = lax.dot_general(z_ref[...], Win_ref[...], (((1,), (1,)), ((), ())),
                        preferred_element_type=jnp.float32) + bin_ref[...]
    x = _gat_layer(x, att_bias3, W0_ref[...], a0c1_ref[...], a0c2_ref[...],
                   g0_ref[...], be0_ref[...], True, ones_h)
    x = _gat_layer(x, att_bias3, W1_ref[...], a1c1_ref[...], a1c2_ref[...],
                   g1_ref[...], be1_ref[...], False, ones_h)
    out_ref[...] = x.reshape(B, N, H)


@jax.jit
def _run(z, W_in, b_in, W0, a0, g0, be0, W1, a1, g1, be1):
    d = pl.pallas_call(
        _dist_body,
        out_shape=jax.ShapeDtypeStruct((ROWS, N), jnp.float32),
    )(z)

    adj = _topk_sc(d)

    return pl.pallas_call(
        _gat_body,
        out_shape=jax.ShapeDtypeStruct((B, N, H), jnp.float32),
    )(adj, z.reshape(ROWS, D), W_in, b_in.reshape(1, H),
      W0, a0[0, :H].reshape(H, 1), a0[0, H:].reshape(1, H),
      g0.reshape(1, H), be0.reshape(1, H),
      W1, a1[0, :H].reshape(H, 1), a1[0, H:].reshape(1, H),
      g1.reshape(1, H), be1.reshape(1, H))


def kernel(z, W_in, b_in, W0, a0, g0, be0, W1, a1, g1, be1):
    return _run(z, W_in, b_in, W0, a0, g0, be0, W1, a1, g1, be1)
